# baseline (speedup 1.0000x reference)
import functools

import jax
import jax.numpy as jnp
from jax import lax
from jax.experimental import pallas as pl
from jax.experimental.pallas import tpu as pltpu
from jax.experimental.pallas import tpu_sc as plsc

N = 10000
E = 320000
D = 128
NC = 2
NS = 16
L = 16
NW = NC * NS
EPW = E // NW
CH = 80
NCHUNK = EPW // CH
ROWS_E = E // CH
NPAD = 10240
CHB = 128
EPWP = 10240
NCHKB = EPWP // CHB
NBLK = 5
NCHB = NCHKB // NBLK


def _tc1_body(x_ref, m_ref, ws_ref, wd_ref, avs_ref, avd_ref,
              h_ref, as_ref, ad_ref):
    xd = x_ref[...] * m_ref[...]
    h = jnp.dot(xd, ws_ref[...], preferred_element_type=jnp.float32)
    hd = jnp.dot(xd, wd_ref[...], preferred_element_type=jnp.float32)
    h_ref[...] = h
    as_ref[...] = jnp.sum(h * avs_ref[...], axis=1, keepdims=True)
    ad_ref[...] = jnp.sum(hd * avd_ref[...], axis=1, keepdims=True)


def _tc1(x, mask, W_src, W_dst, att_src, att_dst):
    R = 1000
    return pl.pallas_call(
        _tc1_body,
        grid=(N // R,),
        in_specs=[
            pl.BlockSpec((R, D), lambda i: (i, 0)),
            pl.BlockSpec((R, D), lambda i: (i, 0)),
            pl.BlockSpec((D, D), lambda i: (0, 0)),
            pl.BlockSpec((D, D), lambda i: (0, 0)),
            pl.BlockSpec((1, D), lambda i: (0, 0)),
            pl.BlockSpec((1, D), lambda i: (0, 0)),
        ],
        out_specs=[
            pl.BlockSpec((R, D), lambda i: (i, 0)),
            pl.BlockSpec((R, 1), lambda i: (i, 0)),
            pl.BlockSpec((R, 1), lambda i: (i, 0)),
        ],
        out_shape=[
            jax.ShapeDtypeStruct((N, D), jnp.float32),
            jax.ShapeDtypeStruct((N, 1), jnp.float32),
            jax.ShapeDtypeStruct((N, 1), jnp.float32),
        ],
    )(x, mask, W_src, W_dst, att_src, att_dst)


def _sc_edge_scalar_body(src_hbm, dst_hbm, as_hbm, ad_hbm, ex_hbm, sparts_hbm,
                         asl, adl, srcl, dstl, exl, zb, s_sh):
    cid = lax.axis_index("c")
    sid = lax.axis_index("s")
    wid = cid * NS + sid

    pltpu.sync_copy(as_hbm, asl)
    pltpu.sync_copy(ad_hbm, adl)
    pltpu.sync_copy(src_hbm.at[wid], srcl)
    pltpu.sync_copy(dst_hbm.at[wid], dstl)

    def _zero(i, _):
        zb[pl.ds(i * L, L)] = jnp.zeros((L,), jnp.float32)
        return 0
    lax.fori_loop(0, 640 // L, _zero, 0)
    pltpu.sync_copy(zb, s_sh.at[pl.ds(sid * 640, 640)])
    plsc.subcore_barrier()

    def _chunk(r, _):
        for c in range(CH // L):
            sl = pl.ds(c * L, L)
            isrc = srcl[r, sl]
            idst = dstl[r, sl]
            e = plsc.load_gather(asl, [isrc]) + plsc.load_gather(adl, [idst])
            e = jnp.where(e >= 0.0, e, 0.2 * e)
            exl[r, sl] = jnp.exp(e)
        pltpu.sync_copy(exl.at[r], s_sh.at[dstl.at[r]], add=True)
        return 0
    lax.fori_loop(0, NCHUNK, _chunk, 0)

    pltpu.sync_copy(exl, ex_hbm.at[wid])
    plsc.subcore_barrier()

    @pl.when(sid == 0)
    def _():
        pltpu.sync_copy(s_sh, sparts_hbm.at[cid])


def _sc_alpha_body(dst_hbm, ex_hbm, sparts_hbm, alpha_hbm,
                   sf, s1, dstl, exl, alphal):
    cid = lax.axis_index("c")
    sid = lax.axis_index("s")
    wid = cid * NS + sid

    pltpu.sync_copy(sparts_hbm.at[0], sf)
    pltpu.sync_copy(sparts_hbm.at[1], s1)
    pltpu.sync_copy(dst_hbm.at[wid], dstl)
    pltpu.sync_copy(ex_hbm.at[wid], exl)

    def _sadd(i, _):
        sl = pl.ds(i * L, L)
        sf[sl] = sf[sl] + s1[sl]
        return 0
    lax.fori_loop(0, NPAD // L, _sadd, 0)

    def _chunk(r, _):
        for c in range(CH // L):
            sl = pl.ds(c * L, L)
            sv = plsc.load_gather(sf, [dstl[r, sl]])
            alphal[r, sl] = exl[r, sl] / (sv + 1e-16)
        return 0
    lax.fori_loop(0, NCHUNK, _chunk, 0)

    pltpu.sync_copy(alphal, alpha_hbm.at[wid])


def _sc_edge_rows_body(src_hbm, dst_hbm, ex_hbm, h_hbm, z_hbm, accp_hbm,
                       srcb, dstb, exb, rows0, rows1, rows2, acc_sh,
                       g0, g1, g2, s0, s1, s2):
    cid = lax.axis_index("c")
    sid = lax.axis_index("s")
    wid = cid * NS + sid

    bufs = (rows0, rows1, rows2)
    gsems = (g0, g1, g2)
    ssems = (s0, s1, s2)

    pltpu.sync_copy(z_hbm, acc_sh.at[pl.ds(sid * 640, 640)])
    plsc.subcore_barrier()

    def _gather(r, i):
        pltpu.async_copy(h_hbm.at[srcb.at[r]], bufs[i], gsems[i])

    def _wait_gather(r, i):
        pltpu.make_async_copy(h_hbm.at[srcb.at[r]], bufs[i], gsems[i]).wait()

    def _scatter(r, i):
        pltpu.sync_copy(bufs[i], acc_sh.at[dstb.at[r]], add=True)

    def _wait_scatter(r, i):
        del r, i

    def _compute(r, i):
        buf = bufs[i]
        ri = jnp.full((L,), r, jnp.int32)
        def _grp(g, _):
            e0 = g * L
            for k in range(L):
                e = e0 + k
                wv = plsc.load_gather(exb, [ri, jnp.full((L,), 1, jnp.int32) * e])
                for c in range(D // L):
                    sl = pl.ds(c * L, L)
                    buf[e, sl] = buf[e, sl] * wv
            return 0
        lax.fori_loop(0, CHB // L, _grp, 0)

    def _block(b, _):
        pltpu.sync_copy(src_hbm.at[wid].at[b], srcb)
        pltpu.sync_copy(dst_hbm.at[wid].at[b], dstb)
        pltpu.sync_copy(ex_hbm.at[wid].at[b], exb)

        _gather(0, 0)

        def _pair(g, _):
            r = 2 * g
            _wait_gather(r, 0)
            _gather(r + 1, 1)
            _compute(r, 0)
            _scatter(r, 0)
            _wait_gather(r + 1, 1)
            _gather(r + 2, 0)
            _compute(r + 1, 1)
            _scatter(r + 1, 1)
            return 0
        lax.fori_loop(0, NCHB // 2 - 1, _pair, 0)
        _wait_gather(NCHB - 2, 0)
        _gather(NCHB - 1, 1)
        _compute(NCHB - 2, 0)
        _scatter(NCHB - 2, 0)
        _wait_gather(NCHB - 1, 1)
        _compute(NCHB - 1, 1)
        _scatter(NCHB - 1, 1)
        return 0
    lax.fori_loop(0, NBLK, _block, 0)

    plsc.subcore_barrier()
    pltpu.sync_copy(acc_sh.at[pl.ds(sid * 640, 640)],
                    accp_hbm.at[cid].at[pl.ds(sid * 640, 640)])


def _tc2_body(p0_ref, p1_ref, s_ref, bc_ref, wl_ref, bl_ref, logp_ref):
    o = (p0_ref[...] + p1_ref[...]) / (s_ref[...] + 1e-16) + bc_ref[...]
    o = o + jnp.dot(o, wl_ref[...], preferred_element_type=jnp.float32) \
        + bl_ref[...]
    o = jnp.maximum(o, 0.0)
    m = jnp.max(o, axis=1, keepdims=True)
    lse = m + jnp.log(jnp.sum(jnp.exp(o - m), axis=1, keepdims=True))
    logp_ref[...] = o - lse


def _tc2(p0, p1, s_tot, b_conv, W_lin, b_lin):
    R = 1000
    return pl.pallas_call(
        _tc2_body,
        grid=(N // R,),
        in_specs=[
            pl.BlockSpec((R, D), lambda i: (i, 0)),
            pl.BlockSpec((R, D), lambda i: (i, 0)),
            pl.BlockSpec((R, 1), lambda i: (i, 0)),
            pl.BlockSpec((1, D), lambda i: (0, 0)),
            pl.BlockSpec((D, D), lambda i: (0, 0)),
            pl.BlockSpec((1, D), lambda i: (0, 0)),
        ],
        out_specs=pl.BlockSpec((R, D), lambda i: (i, 0)),
        out_shape=jax.ShapeDtypeStruct((N, D), jnp.float32),
    )(p0, p1, s_tot, b_conv, W_lin, b_lin)


_sc_cache = {}


def _sc_kernels():
    if "k" not in _sc_cache:
        mesh = plsc.VectorSubcoreMesh(core_axis_name="c", subcore_axis_name="s",
                                      num_cores=NC, num_subcores=NS)
        sc_a = pl.kernel(
            _sc_edge_scalar_body,
            out_type=(
                jax.ShapeDtypeStruct((NW, NCHUNK, CH), jnp.float32),
                jax.ShapeDtypeStruct((NC, NPAD), jnp.float32),
            ),
            mesh=mesh,
            scratch_types=[
                pltpu.VMEM((N,), jnp.float32),
                pltpu.VMEM((N,), jnp.float32),
                pltpu.VMEM((NCHUNK, CH), jnp.int32),
                pltpu.VMEM((NCHUNK, CH), jnp.int32),
                pltpu.VMEM((NCHUNK, CH), jnp.float32),
                pltpu.VMEM((640,), jnp.float32),
                pltpu.VMEM_SHARED((NPAD,), jnp.float32),
            ],
            compiler_params=pltpu.CompilerParams(needs_layout_passes=False),
        )
        sc_c = pl.kernel(
            _sc_alpha_body,
            out_type=jax.ShapeDtypeStruct((NW, NCHUNK, CH), jnp.float32),
            mesh=mesh,
            scratch_types=[
                pltpu.VMEM((NPAD,), jnp.float32),
                pltpu.VMEM((NPAD,), jnp.float32),
                pltpu.VMEM((NCHUNK, CH), jnp.int32),
                pltpu.VMEM((NCHUNK, CH), jnp.float32),
                pltpu.VMEM((NCHUNK, CH), jnp.float32),
            ],
            compiler_params=pltpu.CompilerParams(needs_layout_passes=False),
        )
        sc_b = pl.kernel(
            _sc_edge_rows_body,
            out_type=jax.ShapeDtypeStruct((NC, NPAD, D), jnp.float32),
            mesh=mesh,
            scratch_types=[
                pltpu.VMEM((NCHB, CHB), jnp.int32),
                pltpu.VMEM((NCHB, CHB), jnp.int32),
                pltpu.VMEM((NCHB, CHB), jnp.float32),
                pltpu.VMEM((CHB, D), jnp.float32),
                pltpu.VMEM((CHB, D), jnp.float32),
                pltpu.VMEM((CHB, D), jnp.float32),
                pltpu.VMEM_SHARED((NPAD, D), jnp.float32),
                pltpu.SemaphoreType.DMA,
                pltpu.SemaphoreType.DMA,
                pltpu.SemaphoreType.DMA,
                pltpu.SemaphoreType.DMA,
                pltpu.SemaphoreType.DMA,
                pltpu.SemaphoreType.DMA,
            ],
            compiler_params=pltpu.CompilerParams(needs_layout_passes=False),
        )
        _sc_cache["k"] = (sc_a, sc_c, sc_b)
    return _sc_cache["k"]


def kernel(x, edge_index, W_src, W_dst, att_src, att_dst, b_conv, W_lin, b_lin):
    sc_a, sc_c, sc_b = _sc_kernels()
    src_t = edge_index[0].astype(jnp.int32).reshape(NW, EPW)
    dst_t = edge_index[1].astype(jnp.int32).reshape(NW, EPW)
    src = src_t.reshape(NW, NCHUNK, CH)
    dst = dst_t.reshape(NW, NCHUNK, CH)
    pad = ((0, 0), (0, EPWP - EPW))
    src4 = jnp.pad(src_t, pad).reshape(NW, NBLK, NCHB, CHB)
    dst4 = jnp.pad(dst_t, pad).reshape(NW, NBLK, NCHB, CHB)

    keep = jax.random.bernoulli(jax.random.key(42), 0.4, x.shape)
    mask = jnp.where(keep, 2.5, 0.0).astype(jnp.float32)

    h_src, a_s, a_d = _tc1(x, mask, W_src, W_dst,
                           att_src.reshape(1, D), att_dst.reshape(1, D))
    a_s = a_s.reshape(N)
    a_d = a_d.reshape(N)

    ex, s_parts = sc_a(src, dst, a_s, a_d)
    alpha = sc_c(dst, ex, s_parts)
    ex4 = jnp.pad(ex.reshape(NW, EPW), pad).reshape(NW, NBLK, NCHB, CHB)
    zeros = jnp.zeros((640, D), jnp.float32)
    acc_parts = sc_b(src4, dst4, ex4, h_src, zeros)

    s_tot = (s_parts[0, :N] + s_parts[1, :N]).reshape(N, 1)
    logp = _tc2(acc_parts[0, :N], acc_parts[1, :N], s_tot,
                b_conv.reshape(1, D), W_lin, b_lin.reshape(1, D))
    return logp, alpha.reshape(E)


# speedup vs baseline: 1.8325x; 1.8325x over previous
import functools

import jax
import jax.numpy as jnp
from jax import lax
from jax.experimental import pallas as pl
from jax.experimental.pallas import tpu as pltpu
from jax.experimental.pallas import tpu_sc as plsc

N = 10000
E = 320000
D = 128
NC = 2
NS = 16
L = 16
NW = NC * NS
EPW = E // NW
CH = 80
NCHUNK = EPW // CH
ROWS_E = E // CH
NPAD = 10240
CHB = 80
NCHKB = EPW // CHB
NBLK = 5
NCHB = NCHKB // NBLK


def _tc1_body(x_ref, m_ref, ws_ref, wd_ref, avs_ref, avd_ref,
              h_ref, as_ref, ad_ref):
    xd = x_ref[...] * m_ref[...]
    h = jnp.dot(xd, ws_ref[...], preferred_element_type=jnp.float32)
    hd = jnp.dot(xd, wd_ref[...], preferred_element_type=jnp.float32)
    h_ref[...] = h
    as_ref[...] = jnp.sum(h * avs_ref[...], axis=1, keepdims=True)
    ad_ref[...] = jnp.sum(hd * avd_ref[...], axis=1, keepdims=True)


def _tc1(x, mask, W_src, W_dst, att_src, att_dst):
    R = 1000
    return pl.pallas_call(
        _tc1_body,
        grid=(N // R,),
        in_specs=[
            pl.BlockSpec((R, D), lambda i: (i, 0)),
            pl.BlockSpec((R, D), lambda i: (i, 0)),
            pl.BlockSpec((D, D), lambda i: (0, 0)),
            pl.BlockSpec((D, D), lambda i: (0, 0)),
            pl.BlockSpec((1, D), lambda i: (0, 0)),
            pl.BlockSpec((1, D), lambda i: (0, 0)),
        ],
        out_specs=[
            pl.BlockSpec((R, D), lambda i: (i, 0)),
            pl.BlockSpec((R, 1), lambda i: (i, 0)),
            pl.BlockSpec((R, 1), lambda i: (i, 0)),
        ],
        out_shape=[
            jax.ShapeDtypeStruct((N, D), jnp.float32),
            jax.ShapeDtypeStruct((N, 1), jnp.float32),
            jax.ShapeDtypeStruct((N, 1), jnp.float32),
        ],
    )(x, mask, W_src, W_dst, att_src, att_dst)


def _sc_edge_scalar_body(src_hbm, dst_hbm, as_hbm, ad_hbm, ex_hbm, sparts_hbm,
                         asl, adl, srcl, dstl, exl, zb, s_sh):
    cid = lax.axis_index("c")
    sid = lax.axis_index("s")
    wid = cid * NS + sid

    pltpu.sync_copy(as_hbm, asl)
    pltpu.sync_copy(ad_hbm, adl)
    pltpu.sync_copy(src_hbm.at[wid], srcl)
    pltpu.sync_copy(dst_hbm.at[wid], dstl)

    def _zero(i, _):
        zb[pl.ds(i * L, L)] = jnp.zeros((L,), jnp.float32)
        return 0
    lax.fori_loop(0, 640 // L, _zero, 0)
    pltpu.sync_copy(zb, s_sh.at[pl.ds(sid * 640, 640)])
    plsc.subcore_barrier()

    def _chunk(r, _):
        for c in range(CH // L):
            sl = pl.ds(c * L, L)
            isrc = srcl[r, sl]
            idst = dstl[r, sl]
            e = plsc.load_gather(asl, [isrc]) + plsc.load_gather(adl, [idst])
            e = jnp.where(e >= 0.0, e, 0.2 * e)
            exl[r, sl] = jnp.exp(e)
        pltpu.sync_copy(exl.at[r], s_sh.at[dstl.at[r]], add=True)
        return 0
    lax.fori_loop(0, NCHUNK, _chunk, 0)

    pltpu.sync_copy(exl, ex_hbm.at[wid])
    plsc.subcore_barrier()

    @pl.when(sid == 0)
    def _():
        pltpu.sync_copy(s_sh, sparts_hbm.at[cid])


def _sc_alpha_body(dst_hbm, ex_hbm, sparts_hbm, alpha_hbm,
                   sf, s1, dstl, exl, alphal):
    cid = lax.axis_index("c")
    sid = lax.axis_index("s")
    wid = cid * NS + sid

    pltpu.sync_copy(sparts_hbm.at[0], sf)
    pltpu.sync_copy(sparts_hbm.at[1], s1)
    pltpu.sync_copy(dst_hbm.at[wid], dstl)
    pltpu.sync_copy(ex_hbm.at[wid], exl)

    def _sadd(i, _):
        sl = pl.ds(i * L, L)
        sf[sl] = sf[sl] + s1[sl]
        return 0
    lax.fori_loop(0, NPAD // L, _sadd, 0)

    def _chunk(r, _):
        for c in range(CH // L):
            sl = pl.ds(c * L, L)
            sv = plsc.load_gather(sf, [dstl[r, sl]])
            alphal[r, sl] = exl[r, sl] / (sv + 1e-16)
        return 0
    lax.fori_loop(0, NCHUNK, _chunk, 0)

    pltpu.sync_copy(alphal, alpha_hbm.at[wid])


def _sc_edge_rows_body(src_hbm, dst_hbm, ex_hbm, h_hbm, z_hbm, accp_hbm,
                       srcb, dstb, exb, rows0, rows1, rows2, acc_sh,
                       g0, g1, g2, x0, x1, x2):
    cid = lax.axis_index("c")
    sid = lax.axis_index("s")
    wid = cid * NS + sid

    bufs = (rows0, rows1, rows2)
    gsems = (g0, g1, g2)

    pltpu.sync_copy(z_hbm, acc_sh.at[pl.ds(sid * 640, 640)])
    plsc.subcore_barrier()

    def _gather(r, i):
        pltpu.async_copy(h_hbm.at[srcb.at[r]], bufs[i], gsems[i])

    def _wait_gather(r, i):
        pltpu.make_async_copy(h_hbm.at[srcb.at[r]], bufs[i], gsems[i]).wait()

    def _scatter(r, i):
        pltpu.sync_copy(bufs[i], acc_sh.at[dstb.at[r]], add=True)

    def _wait_scatter(r, i):
        del r, i

    def _compute(r, i):
        buf = bufs[i]
        ri = jnp.full((L,), r, jnp.int32)
        def _grp(g, _):
            e0 = g * L
            for k in range(L):
                e = e0 + k
                wv = plsc.load_gather(exb, [ri, jnp.full((L,), 1, jnp.int32) * e])
                for c in range(D // L):
                    sl = pl.ds(c * L, L)
                    buf[e, sl] = buf[e, sl] * wv
            return 0
        lax.fori_loop(0, CHB // L, _grp, 0)

    def _proc(r, i, prefetch):
        _wait_gather(r, i)
        if prefetch:
            _gather(r + 2, (i + 2) % 3)
        _compute(r, i)
        _scatter(r, i)

    def _block(b, _):
        pltpu.sync_copy(src_hbm.at[wid].at[b], srcb)
        pltpu.sync_copy(dst_hbm.at[wid].at[b], dstb)
        pltpu.sync_copy(ex_hbm.at[wid].at[b], exb)

        _gather(0, 0)
        _gather(1, 1)
        _proc(0, 0, True)
        _proc(1, 1, True)

        def _triple(t, _):
            r = 3 * t + 2
            _proc(r, 2, True)
            _proc(r + 1, 0, True)
            _proc(r + 2, 1, True)
            return 0
        lax.fori_loop(0, (NCHB - 4) // 3, _triple, 0)

        _proc(NCHB - 2, (NCHB - 2) % 3, False)
        _proc(NCHB - 1, (NCHB - 1) % 3, False)
        return 0
    lax.fori_loop(0, NBLK, _block, 0)

    plsc.subcore_barrier()
    pltpu.sync_copy(acc_sh.at[pl.ds(sid * 640, 640)],
                    accp_hbm.at[cid].at[pl.ds(sid * 640, 640)])


def _tc2_body(p0_ref, p1_ref, s_ref, bc_ref, wl_ref, bl_ref, logp_ref):
    o = (p0_ref[...] + p1_ref[...]) / (s_ref[...] + 1e-16) + bc_ref[...]
    o = o + jnp.dot(o, wl_ref[...], preferred_element_type=jnp.float32) \
        + bl_ref[...]
    o = jnp.maximum(o, 0.0)
    m = jnp.max(o, axis=1, keepdims=True)
    lse = m + jnp.log(jnp.sum(jnp.exp(o - m), axis=1, keepdims=True))
    logp_ref[...] = o - lse


def _tc2(p0, p1, s_tot, b_conv, W_lin, b_lin):
    R = 1000
    return pl.pallas_call(
        _tc2_body,
        grid=(N // R,),
        in_specs=[
            pl.BlockSpec((R, D), lambda i: (i, 0)),
            pl.BlockSpec((R, D), lambda i: (i, 0)),
            pl.BlockSpec((R, 1), lambda i: (i, 0)),
            pl.BlockSpec((1, D), lambda i: (0, 0)),
            pl.BlockSpec((D, D), lambda i: (0, 0)),
            pl.BlockSpec((1, D), lambda i: (0, 0)),
        ],
        out_specs=pl.BlockSpec((R, D), lambda i: (i, 0)),
        out_shape=jax.ShapeDtypeStruct((N, D), jnp.float32),
    )(p0, p1, s_tot, b_conv, W_lin, b_lin)


_sc_cache = {}


def _sc_kernels():
    if "k" not in _sc_cache:
        mesh = plsc.VectorSubcoreMesh(core_axis_name="c", subcore_axis_name="s",
                                      num_cores=NC, num_subcores=NS)
        sc_a = pl.kernel(
            _sc_edge_scalar_body,
            out_type=(
                jax.ShapeDtypeStruct((NW, NCHUNK, CH), jnp.float32),
                jax.ShapeDtypeStruct((NC, NPAD), jnp.float32),
            ),
            mesh=mesh,
            scratch_types=[
                pltpu.VMEM((N,), jnp.float32),
                pltpu.VMEM((N,), jnp.float32),
                pltpu.VMEM((NCHUNK, CH), jnp.int32),
                pltpu.VMEM((NCHUNK, CH), jnp.int32),
                pltpu.VMEM((NCHUNK, CH), jnp.float32),
                pltpu.VMEM((640,), jnp.float32),
                pltpu.VMEM_SHARED((NPAD,), jnp.float32),
            ],
            compiler_params=pltpu.CompilerParams(needs_layout_passes=False),
        )
        sc_c = pl.kernel(
            _sc_alpha_body,
            out_type=jax.ShapeDtypeStruct((NW, NCHUNK, CH), jnp.float32),
            mesh=mesh,
            scratch_types=[
                pltpu.VMEM((NPAD,), jnp.float32),
                pltpu.VMEM((NPAD,), jnp.float32),
                pltpu.VMEM((NCHUNK, CH), jnp.int32),
                pltpu.VMEM((NCHUNK, CH), jnp.float32),
                pltpu.VMEM((NCHUNK, CH), jnp.float32),
            ],
            compiler_params=pltpu.CompilerParams(needs_layout_passes=False),
        )
        sc_b = pl.kernel(
            _sc_edge_rows_body,
            out_type=jax.ShapeDtypeStruct((NC, NPAD, D), jnp.float32),
            mesh=mesh,
            scratch_types=[
                pltpu.VMEM((NCHB, CHB), jnp.int32),
                pltpu.VMEM((NCHB, CHB), jnp.int32),
                pltpu.VMEM((NCHB, CHB), jnp.float32),
                pltpu.VMEM((CHB, D), jnp.float32),
                pltpu.VMEM((CHB, D), jnp.float32),
                pltpu.VMEM((CHB, D), jnp.float32),
                pltpu.VMEM_SHARED((NPAD, D), jnp.float32),
                pltpu.SemaphoreType.DMA,
                pltpu.SemaphoreType.DMA,
                pltpu.SemaphoreType.DMA,
                pltpu.SemaphoreType.DMA,
                pltpu.SemaphoreType.DMA,
                pltpu.SemaphoreType.DMA,
            ],
            compiler_params=pltpu.CompilerParams(needs_layout_passes=False),
        )
        _sc_cache["k"] = (sc_a, sc_c, sc_b)
    return _sc_cache["k"]


def kernel(x, edge_index, W_src, W_dst, att_src, att_dst, b_conv, W_lin, b_lin):
    sc_a, sc_c, sc_b = _sc_kernels()
    src = edge_index[0].astype(jnp.int32).reshape(NW, NCHUNK, CH)
    dst = edge_index[1].astype(jnp.int32).reshape(NW, NCHUNK, CH)
    src4 = src.reshape(NW, NBLK, NCHB, CHB)
    dst4 = dst.reshape(NW, NBLK, NCHB, CHB)

    keep = jax.random.bernoulli(jax.random.key(42), 0.4, x.shape)
    mask = jnp.where(keep, 2.5, 0.0).astype(jnp.float32)

    h_src, a_s, a_d = _tc1(x, mask, W_src, W_dst,
                           att_src.reshape(1, D), att_dst.reshape(1, D))
    a_s = a_s.reshape(N)
    a_d = a_d.reshape(N)

    ex, s_parts = sc_a(src, dst, a_s, a_d)
    alpha = sc_c(dst, ex, s_parts)
    ex4 = ex.reshape(NW, NBLK, NCHB, CHB)
    zeros = jnp.zeros((640, D), jnp.float32)
    acc_parts = sc_b(src4, dst4, ex4, h_src, zeros)

    s_tot = (s_parts[0, :N] + s_parts[1, :N]).reshape(N, 1)
    logp = _tc2(acc_parts[0, :N], acc_parts[1, :N], s_tot,
                b_conv.reshape(1, D), W_lin, b_lin.reshape(1, D))
    return logp, alpha.reshape(E)


# speedup vs baseline: 1.8498x; 1.0095x over previous
import functools

import jax
import jax.numpy as jnp
from jax import lax
from jax.experimental import pallas as pl
from jax.experimental.pallas import tpu as pltpu
from jax.experimental.pallas import tpu_sc as plsc

N = 10000
E = 320000
D = 128
NC = 2
NS = 16
L = 16
NW = NC * NS
EPW = E // NW
CH = 80
NCHUNK = EPW // CH
ROWS_E = E // CH
NPAD = 10240
CHB = 80
NCHKB = EPW // CHB
NBLK = 5
NCHB = NCHKB // NBLK


def _tc1_body(x_ref, m_ref, ws_ref, wd_ref, avs_ref, avd_ref,
              h_ref, as_ref, ad_ref):
    xd = x_ref[...] * m_ref[...]
    h = jnp.dot(xd, ws_ref[...], preferred_element_type=jnp.float32)
    hd = jnp.dot(xd, wd_ref[...], preferred_element_type=jnp.float32)
    h_ref[...] = h
    as_ref[...] = jnp.sum(h * avs_ref[...], axis=1, keepdims=True)
    ad_ref[...] = jnp.sum(hd * avd_ref[...], axis=1, keepdims=True)


def _tc1(x, mask, W_src, W_dst, att_src, att_dst):
    R = 1000
    return pl.pallas_call(
        _tc1_body,
        grid=(N // R,),
        in_specs=[
            pl.BlockSpec((R, D), lambda i: (i, 0)),
            pl.BlockSpec((R, D), lambda i: (i, 0)),
            pl.BlockSpec((D, D), lambda i: (0, 0)),
            pl.BlockSpec((D, D), lambda i: (0, 0)),
            pl.BlockSpec((1, D), lambda i: (0, 0)),
            pl.BlockSpec((1, D), lambda i: (0, 0)),
        ],
        out_specs=[
            pl.BlockSpec((R, D), lambda i: (i, 0)),
            pl.BlockSpec((R, 1), lambda i: (i, 0)),
            pl.BlockSpec((R, 1), lambda i: (i, 0)),
        ],
        out_shape=[
            jax.ShapeDtypeStruct((N, D), jnp.float32),
            jax.ShapeDtypeStruct((N, 1), jnp.float32),
            jax.ShapeDtypeStruct((N, 1), jnp.float32),
        ],
    )(x, mask, W_src, W_dst, att_src, att_dst)


def _sc_edge_scalar_body(src_hbm, dst_hbm, as_hbm, ad_hbm, ex_hbm, sparts_hbm,
                         asl, adl, srcl, dstl, exl, zb, s_sh):
    cid = lax.axis_index("c")
    sid = lax.axis_index("s")
    wid = cid * NS + sid

    pltpu.sync_copy(as_hbm, asl)
    pltpu.sync_copy(ad_hbm, adl)
    pltpu.sync_copy(src_hbm.at[wid], srcl)
    pltpu.sync_copy(dst_hbm.at[wid], dstl)

    def _zero(i, _):
        zb[pl.ds(i * L, L)] = jnp.zeros((L,), jnp.float32)
        return 0
    lax.fori_loop(0, 640 // L, _zero, 0)
    pltpu.sync_copy(zb, s_sh.at[pl.ds(sid * 640, 640)])
    plsc.subcore_barrier()

    def _chunk(r, _):
        for c in range(CH // L):
            sl = pl.ds(c * L, L)
            isrc = srcl[r, sl]
            idst = dstl[r, sl]
            e = plsc.load_gather(asl, [isrc]) + plsc.load_gather(adl, [idst])
            e = jnp.where(e >= 0.0, e, 0.2 * e)
            exl[r, sl] = jnp.exp(e)
        pltpu.sync_copy(exl.at[r], s_sh.at[dstl.at[r]], add=True)
        return 0
    lax.fori_loop(0, NCHUNK, _chunk, 0)

    pltpu.sync_copy(exl, ex_hbm.at[wid])
    plsc.subcore_barrier()

    @pl.when(sid == 0)
    def _():
        pltpu.sync_copy(s_sh, sparts_hbm.at[cid])


def _sc_alpha_body(dst_hbm, ex_hbm, sparts_hbm, alpha_hbm,
                   sf, s1, dstl, exl, alphal):
    cid = lax.axis_index("c")
    sid = lax.axis_index("s")
    wid = cid * NS + sid

    pltpu.sync_copy(sparts_hbm.at[0], sf)
    pltpu.sync_copy(sparts_hbm.at[1], s1)
    pltpu.sync_copy(dst_hbm.at[wid], dstl)
    pltpu.sync_copy(ex_hbm.at[wid], exl)

    def _sadd(i, _):
        sl = pl.ds(i * L, L)
        sf[sl] = sf[sl] + s1[sl]
        return 0
    lax.fori_loop(0, NPAD // L, _sadd, 0)

    def _chunk(r, _):
        for c in range(CH // L):
            sl = pl.ds(c * L, L)
            sv = plsc.load_gather(sf, [dstl[r, sl]])
            alphal[r, sl] = exl[r, sl] / (sv + 1e-16)
        return 0
    lax.fori_loop(0, NCHUNK, _chunk, 0)

    pltpu.sync_copy(alphal, alpha_hbm.at[wid])


def _sc_edge_rows_body(src_hbm, dst_hbm, ex_hbm, h_hbm, z_hbm, accp_hbm,
                       srcb, dstb, exb, rows0, rows1, rows2, acc_sh,
                       g0, g1, g2, x0, x1, x2):
    cid = lax.axis_index("c")
    sid = lax.axis_index("s")
    wid = cid * NS + sid

    bufs = (rows0, rows1, rows2)
    gsems = (g0, g1, g2)

    pltpu.sync_copy(z_hbm, acc_sh.at[pl.ds(sid * 640, 640)])
    plsc.subcore_barrier()

    def _gather(r, i):
        pltpu.async_copy(h_hbm.at[srcb.at[r]], bufs[i], gsems[i])

    def _wait_gather(r, i):
        pltpu.make_async_copy(h_hbm.at[srcb.at[r]], bufs[i], gsems[i]).wait()

    def _scatter(r, i):
        pltpu.sync_copy(bufs[i], acc_sh.at[dstb.at[r]], add=True)

    def _wait_scatter(r, i):
        del r, i

    def _compute(r, i):
        buf = bufs[i]
        ri = jnp.full((L,), r, jnp.int32)
        def _grp(g, _):
            e0 = g * L
            for k in range(L):
                e = e0 + k
                wv = plsc.load_gather(exb, [ri, jnp.full((L,), 1, jnp.int32) * e])
                for c in range(D // L):
                    sl = pl.ds(c * L, L)
                    buf[e, sl] = buf[e, sl] * wv
            return 0
        lax.fori_loop(0, CHB // L, _grp, 0)

    def _block(b, _):
        pltpu.sync_copy(src_hbm.at[wid].at[b], srcb)
        pltpu.sync_copy(dst_hbm.at[wid].at[b], dstb)
        pltpu.sync_copy(ex_hbm.at[wid].at[b], exb)

        _gather(0, 0)

        def _pair(g, _):
            r = 2 * g
            _wait_gather(r, 0)
            _gather(r + 1, 1)
            _compute(r, 0)
            _scatter(r, 0)
            _wait_gather(r + 1, 1)
            _gather(r + 2, 0)
            _compute(r + 1, 1)
            _scatter(r + 1, 1)
            return 0
        lax.fori_loop(0, (NCHB - 1) // 2, _pair, 0)
        _wait_gather(NCHB - 1, 0)
        _compute(NCHB - 1, 0)
        _scatter(NCHB - 1, 0)
        return 0
    lax.fori_loop(0, NBLK, _block, 0)

    plsc.subcore_barrier()
    pltpu.sync_copy(acc_sh.at[pl.ds(sid * 640, 640)],
                    accp_hbm.at[cid].at[pl.ds(sid * 640, 640)])


def _tc2_body(p0_ref, p1_ref, s_ref, bc_ref, wl_ref, bl_ref, logp_ref):
    o = (p0_ref[...] + p1_ref[...]) / (s_ref[...] + 1e-16) + bc_ref[...]
    o = o + jnp.dot(o, wl_ref[...], preferred_element_type=jnp.float32) \
        + bl_ref[...]
    o = jnp.maximum(o, 0.0)
    m = jnp.max(o, axis=1, keepdims=True)
    lse = m + jnp.log(jnp.sum(jnp.exp(o - m), axis=1, keepdims=True))
    logp_ref[...] = o - lse


def _tc2(p0, p1, s_tot, b_conv, W_lin, b_lin):
    R = 1000
    return pl.pallas_call(
        _tc2_body,
        grid=(N // R,),
        in_specs=[
            pl.BlockSpec((R, D), lambda i: (i, 0)),
            pl.BlockSpec((R, D), lambda i: (i, 0)),
            pl.BlockSpec((R, 1), lambda i: (i, 0)),
            pl.BlockSpec((1, D), lambda i: (0, 0)),
            pl.BlockSpec((D, D), lambda i: (0, 0)),
            pl.BlockSpec((1, D), lambda i: (0, 0)),
        ],
        out_specs=pl.BlockSpec((R, D), lambda i: (i, 0)),
        out_shape=jax.ShapeDtypeStruct((N, D), jnp.float32),
    )(p0, p1, s_tot, b_conv, W_lin, b_lin)


_sc_cache = {}


def _sc_kernels():
    if "k" not in _sc_cache:
        mesh = plsc.VectorSubcoreMesh(core_axis_name="c", subcore_axis_name="s",
                                      num_cores=NC, num_subcores=NS)
        sc_a = pl.kernel(
            _sc_edge_scalar_body,
            out_type=(
                jax.ShapeDtypeStruct((NW, NCHUNK, CH), jnp.float32),
                jax.ShapeDtypeStruct((NC, NPAD), jnp.float32),
            ),
            mesh=mesh,
            scratch_types=[
                pltpu.VMEM((N,), jnp.float32),
                pltpu.VMEM((N,), jnp.float32),
                pltpu.VMEM((NCHUNK, CH), jnp.int32),
                pltpu.VMEM((NCHUNK, CH), jnp.int32),
                pltpu.VMEM((NCHUNK, CH), jnp.float32),
                pltpu.VMEM((640,), jnp.float32),
                pltpu.VMEM_SHARED((NPAD,), jnp.float32),
            ],
            compiler_params=pltpu.CompilerParams(needs_layout_passes=False),
        )
        sc_c = pl.kernel(
            _sc_alpha_body,
            out_type=jax.ShapeDtypeStruct((NW, NCHUNK, CH), jnp.float32),
            mesh=mesh,
            scratch_types=[
                pltpu.VMEM((NPAD,), jnp.float32),
                pltpu.VMEM((NPAD,), jnp.float32),
                pltpu.VMEM((NCHUNK, CH), jnp.int32),
                pltpu.VMEM((NCHUNK, CH), jnp.float32),
                pltpu.VMEM((NCHUNK, CH), jnp.float32),
            ],
            compiler_params=pltpu.CompilerParams(needs_layout_passes=False),
        )
        sc_b = pl.kernel(
            _sc_edge_rows_body,
            out_type=jax.ShapeDtypeStruct((NC, NPAD, D), jnp.float32),
            mesh=mesh,
            scratch_types=[
                pltpu.VMEM((NCHB, CHB), jnp.int32),
                pltpu.VMEM((NCHB, CHB), jnp.int32),
                pltpu.VMEM((NCHB, CHB), jnp.float32),
                pltpu.VMEM((CHB, D), jnp.float32),
                pltpu.VMEM((CHB, D), jnp.float32),
                pltpu.VMEM((CHB, D), jnp.float32),
                pltpu.VMEM_SHARED((NPAD, D), jnp.float32),
                pltpu.SemaphoreType.DMA,
                pltpu.SemaphoreType.DMA,
                pltpu.SemaphoreType.DMA,
                pltpu.SemaphoreType.DMA,
                pltpu.SemaphoreType.DMA,
                pltpu.SemaphoreType.DMA,
            ],
            compiler_params=pltpu.CompilerParams(needs_layout_passes=False),
        )
        _sc_cache["k"] = (sc_a, sc_c, sc_b)
    return _sc_cache["k"]


def kernel(x, edge_index, W_src, W_dst, att_src, att_dst, b_conv, W_lin, b_lin):
    sc_a, sc_c, sc_b = _sc_kernels()
    src = edge_index[0].astype(jnp.int32).reshape(NW, NCHUNK, CH)
    dst = edge_index[1].astype(jnp.int32).reshape(NW, NCHUNK, CH)
    src4 = src.reshape(NW, NBLK, NCHB, CHB)
    dst4 = dst.reshape(NW, NBLK, NCHB, CHB)

    keep = jax.random.bernoulli(jax.random.key(42), 0.4, x.shape)
    mask = jnp.where(keep, 2.5, 0.0).astype(jnp.float32)

    h_src, a_s, a_d = _tc1(x, mask, W_src, W_dst,
                           att_src.reshape(1, D), att_dst.reshape(1, D))
    a_s = a_s.reshape(N)
    a_d = a_d.reshape(N)

    ex, s_parts = sc_a(src, dst, a_s, a_d)
    alpha = sc_c(dst, ex, s_parts)
    ex4 = ex.reshape(NW, NBLK, NCHB, CHB)
    zeros = jnp.zeros((640, D), jnp.float32)
    acc_parts = sc_b(src4, dst4, ex4, h_src, zeros)

    s_tot = (s_parts[0, :N] + s_parts[1, :N]).reshape(N, 1)
    logp = _tc2(acc_parts[0, :N], acc_parts[1, :N], s_tot,
                b_conv.reshape(1, D), W_lin, b_lin.reshape(1, D))
    return logp, alpha.reshape(E)
